# Initial kernel scaffold; baseline (speedup 1.0000x reference)
import jax
import jax.numpy as jnp
from jax.experimental import pallas as pl


def kernel(x, w_xy, w_xh, w_hy, k_y, k_h):
    raise NotImplementedError("write your pallas kernel here")


# speedup vs baseline: 11.3160x; 11.3160x over previous
import functools

import jax
import jax.numpy as jnp
from jax.experimental import pallas as pl


def _kth_threshold(a, k, iters):
    lo = jnp.min(a, axis=-1, keepdims=True)
    hi = jnp.max(a, axis=-1, keepdims=True)
    for _ in range(iters):
        mid = jnp.floor((lo + hi + 1.0) * 0.5)
        cnt = jnp.sum((a >= mid).astype(jnp.float32), axis=-1, keepdims=True)
        ge = cnt >= k
        lo = jnp.where(ge, mid, lo)
        hi = jnp.where(ge, hi, mid - 1.0)
    return lo


def _wta_block(x_ref, wxy_ref, wxh_ref, why_ref, y_ref):
    x = x_ref[...].astype(jnp.bfloat16)
    a_h = jnp.dot(x, wxh_ref[...], preferred_element_type=jnp.float32)
    thr_h = _kth_threshold(a_h, 13, iters=11)
    h = jnp.where(a_h >= thr_h, a_h, 0.0).astype(jnp.bfloat16)
    a_y = jnp.dot(x, wxy_ref[...], preferred_element_type=jnp.float32)
    a_y = a_y - jnp.dot(h, why_ref[...], preferred_element_type=jnp.float32)
    thr_y = _kth_threshold(a_y, 51, iters=19)
    y_ref[...] = jnp.where(a_y >= thr_y, a_y, 0.0)


@jax.jit
def _wta(x, w_xy, w_xh, w_hy):
    B, NX = x.shape
    NY = w_xy.shape[1]
    NH = w_xh.shape[1]
    BLK = 512
    grid = (B // BLK,)
    return pl.pallas_call(
        _wta_block,
        grid=grid,
        in_specs=[
            pl.BlockSpec((BLK, NX), lambda i: (i, 0)),
            pl.BlockSpec((NX, NY), lambda i: (0, 0)),
            pl.BlockSpec((NX, NH), lambda i: (0, 0)),
            pl.BlockSpec((NH, NY), lambda i: (0, 0)),
        ],
        out_specs=pl.BlockSpec((BLK, NY), lambda i: (i, 0)),
        out_shape=jax.ShapeDtypeStruct((B, NY), jnp.float32),
    )(x, w_xy, w_xh, w_hy)


def kernel(x, w_xy, w_xh, w_hy, k_y, k_h):
    return _wta(
        x,
        w_xy.astype(jnp.bfloat16),
        w_xh.astype(jnp.bfloat16),
        w_hy.astype(jnp.bfloat16),
    )


# speedup vs baseline: 12.8535x; 1.1359x over previous
import functools

import jax
import jax.numpy as jnp
from jax.experimental import pallas as pl


def _kth_threshold(a, k):
    lo = jnp.min(a, axis=-1, keepdims=True)
    hi = jnp.max(a, axis=-1, keepdims=True)

    def cond(carry):
        lo, hi = carry
        return jnp.max(hi - lo) > 0.0

    def body(carry):
        lo, hi = carry
        mid = jnp.floor((lo + hi + 1.0) * 0.5)
        cnt = jnp.sum((a >= mid).astype(jnp.float32), axis=-1, keepdims=True)
        ge = cnt >= k
        return jnp.where(ge, mid, lo), jnp.where(ge, hi, mid - 1.0)

    lo, hi = jax.lax.while_loop(cond, body, (lo, hi))
    return lo


def _wta_block(x_ref, wxy_ref, wxh_ref, why_ref, y_ref):
    x = x_ref[...].astype(jnp.bfloat16)
    a_h = jnp.dot(x, wxh_ref[...], preferred_element_type=jnp.float32)
    thr_h = _kth_threshold(a_h, 13)
    h = jnp.where(a_h >= thr_h, a_h, 0.0).astype(jnp.bfloat16)
    a_y = jnp.dot(x, wxy_ref[...], preferred_element_type=jnp.float32)
    a_y = a_y - jnp.dot(h, why_ref[...], preferred_element_type=jnp.float32)
    thr_y = _kth_threshold(a_y, 51)
    y_ref[...] = jnp.where(a_y >= thr_y, a_y, 0.0)


@jax.jit
def _wta(x, w_xy, w_xh, w_hy):
    B, NX = x.shape
    NY = w_xy.shape[1]
    NH = w_xh.shape[1]
    BLK = 512
    grid = (B // BLK,)
    return pl.pallas_call(
        _wta_block,
        grid=grid,
        in_specs=[
            pl.BlockSpec((BLK, NX), lambda i: (i, 0)),
            pl.BlockSpec((NX, NY), lambda i: (0, 0)),
            pl.BlockSpec((NX, NH), lambda i: (0, 0)),
            pl.BlockSpec((NH, NY), lambda i: (0, 0)),
        ],
        out_specs=pl.BlockSpec((BLK, NY), lambda i: (i, 0)),
        out_shape=jax.ShapeDtypeStruct((B, NY), jnp.float32),
    )(x, w_xy, w_xh, w_hy)


def kernel(x, w_xy, w_xh, w_hy, k_y, k_h):
    return _wta(
        x,
        w_xy.astype(jnp.bfloat16),
        w_xh.astype(jnp.bfloat16),
        w_hy.astype(jnp.bfloat16),
    )


# speedup vs baseline: 13.4388x; 1.0455x over previous
import functools

import jax
import jax.numpy as jnp
from jax.experimental import pallas as pl


def _kth_threshold(a, k):
    lo = jnp.min(a, axis=-1, keepdims=True)
    hi = jnp.max(a, axis=-1, keepdims=True)

    def cond(carry):
        lo, hi = carry
        return jnp.max(hi - lo) > 0.0

    def body(carry):
        lo, hi = carry
        mid = jnp.floor((lo + hi + 1.0) * 0.5)
        cnt = jnp.sum((a >= mid).astype(jnp.float32), axis=-1, keepdims=True)
        ge = cnt >= k
        return jnp.where(ge, mid, lo), jnp.where(ge, hi, mid - 1.0)

    lo, hi = jax.lax.while_loop(cond, body, (lo, hi))
    return lo


def _wta_block(x_ref, wxy_ref, wxh_ref, why_ref, y_ref):
    x = x_ref[...].astype(jnp.bfloat16)
    a_h = jnp.dot(x, wxh_ref[...], preferred_element_type=jnp.float32)
    thr_h = _kth_threshold(a_h, 13)
    h = jnp.where(a_h >= thr_h, a_h, 0.0).astype(jnp.bfloat16)
    a_y = jnp.dot(x, wxy_ref[...], preferred_element_type=jnp.float32)
    a_y = a_y - jnp.dot(h, why_ref[...], preferred_element_type=jnp.float32)
    thr_y = _kth_threshold(a_y, 51)
    y_ref[...] = jnp.where(a_y >= thr_y, a_y, 0.0)


@jax.jit
def _wta(x, w_xy, w_xh, w_hy):
    B, NX = x.shape
    NY = w_xy.shape[1]
    NH = w_xh.shape[1]
    BLK = 1024
    grid = (B // BLK,)
    return pl.pallas_call(
        _wta_block,
        grid=grid,
        in_specs=[
            pl.BlockSpec((BLK, NX), lambda i: (i, 0)),
            pl.BlockSpec((NX, NY), lambda i: (0, 0)),
            pl.BlockSpec((NX, NH), lambda i: (0, 0)),
            pl.BlockSpec((NH, NY), lambda i: (0, 0)),
        ],
        out_specs=pl.BlockSpec((BLK, NY), lambda i: (i, 0)),
        out_shape=jax.ShapeDtypeStruct((B, NY), jnp.float32),
    )(x, w_xy, w_xh, w_hy)


def kernel(x, w_xy, w_xh, w_hy, k_y, k_h):
    return _wta(
        x,
        w_xy.astype(jnp.bfloat16),
        w_xh.astype(jnp.bfloat16),
        w_hy.astype(jnp.bfloat16),
    )


# speedup vs baseline: 14.6458x; 1.0898x over previous
import jax
import jax.numpy as jnp
from jax.experimental import pallas as pl
from jax.experimental.pallas import tpu as pltpu


def _kth_threshold(a, k, lo, hi):
    def cond(carry):
        lo, hi = carry
        return jnp.max(hi - lo) > 0.0

    def body(carry):
        lo, hi = carry
        mid = jnp.floor((lo + hi + 1.0) * 0.5)
        cnt = jnp.sum((a >= mid).astype(jnp.float32), axis=-1, keepdims=True)
        ge = cnt >= k
        return jnp.where(ge, mid, lo), jnp.where(ge, hi, mid - 1.0)

    lo, hi = jax.lax.while_loop(cond, body, (lo, hi))
    return lo


def _wta_block(x_ref, wxh_ref, wcat_ref, y_ref, cat_ref):
    x = x_ref[...].astype(jnp.bfloat16)
    a_h = jnp.dot(x, wxh_ref[...], preferred_element_type=jnp.float32)
    thr_h = _kth_threshold(
        a_h, 13, jnp.zeros_like(a_h[:, :1]), jnp.max(a_h, axis=-1, keepdims=True)
    )
    h = jnp.where(a_h >= thr_h, a_h, 0.0).astype(jnp.bfloat16)
    cat_ref[:, : x.shape[1]] = x
    cat_ref[:, x.shape[1] :] = h
    a_y = jnp.dot(cat_ref[...], wcat_ref[...], preferred_element_type=jnp.float32)
    thr_y = _kth_threshold(
        a_y,
        51,
        jnp.min(a_y, axis=-1, keepdims=True),
        jnp.max(a_y, axis=-1, keepdims=True),
    )
    y_ref[...] = jnp.where(a_y >= thr_y, a_y, 0.0)


@jax.jit
def _wta(x, w_xh, w_cat):
    B, NX = x.shape
    NH = w_xh.shape[1]
    NY = w_cat.shape[1]
    BLK = 1024
    grid = (B // BLK,)
    return pl.pallas_call(
        _wta_block,
        grid=grid,
        in_specs=[
            pl.BlockSpec((BLK, NX), lambda i: (i, 0)),
            pl.BlockSpec((NX, NH), lambda i: (0, 0)),
            pl.BlockSpec((NX + NH, NY), lambda i: (0, 0)),
        ],
        out_specs=pl.BlockSpec((BLK, NY), lambda i: (i, 0)),
        out_shape=jax.ShapeDtypeStruct((B, NY), jnp.float32),
        scratch_shapes=[pltpu.VMEM((BLK, NX + NH), jnp.bfloat16)],
    )(x, w_xh, w_cat)


def kernel(x, w_xy, w_xh, w_hy, k_y, k_h):
    w_cat = jnp.concatenate(
        [w_xy.astype(jnp.bfloat16), -w_hy.astype(jnp.bfloat16)], axis=0
    )
    return _wta(x, w_xh.astype(jnp.bfloat16), w_cat)


# speedup vs baseline: 16.8963x; 1.1537x over previous
import jax
import jax.numpy as jnp
from jax.experimental import pallas as pl
from jax.experimental.pallas import tpu as pltpu


def _kth_threshold(a, k, lo, hi):
    def cond(carry):
        lo, hi = carry
        return jnp.max(hi - lo) > 0.0

    def step(lo, hi):
        mid = jnp.floor((lo + hi + 1.0) * 0.5)
        cnt = jnp.sum((a >= mid).astype(jnp.float32), axis=-1, keepdims=True)
        ge = cnt >= k
        return jnp.where(ge, mid, lo), jnp.where(ge, hi, mid - 1.0)

    def body(carry):
        return step(*step(*carry))

    lo, hi = jax.lax.while_loop(cond, body, (lo, hi))
    return lo


def _wta_block(x_ref, wxh_ref, wcat_ref, y_ref, cat_ref):
    x = x_ref[...].astype(jnp.bfloat16)
    a_h = jnp.dot(x, wxh_ref[...], preferred_element_type=jnp.float32)
    thr_h = _kth_threshold(
        a_h, 13, jnp.zeros_like(a_h[:, :1]), jnp.max(a_h, axis=-1, keepdims=True)
    )
    h = jnp.where(a_h >= thr_h, a_h, 0.0).astype(jnp.bfloat16)
    cat_ref[:, : x.shape[1]] = x
    cat_ref[:, x.shape[1] :] = h
    a_y = jnp.dot(cat_ref[...], wcat_ref[...], preferred_element_type=jnp.float32)
    thr_y = _kth_threshold(
        a_y,
        51,
        jnp.min(a_y, axis=-1, keepdims=True),
        jnp.max(a_y, axis=-1, keepdims=True),
    )
    y_ref[...] = jnp.where(a_y >= thr_y, a_y, 0.0)


@jax.jit
def _wta(x, w_xh, w_cat):
    B, NX = x.shape
    NH = w_xh.shape[1]
    NY = w_cat.shape[1]
    BLK = 1024
    grid = (B // BLK,)
    return pl.pallas_call(
        _wta_block,
        grid=grid,
        in_specs=[
            pl.BlockSpec((BLK, NX), lambda i: (i, 0)),
            pl.BlockSpec((NX, NH), lambda i: (0, 0)),
            pl.BlockSpec((NX + NH, NY), lambda i: (0, 0)),
        ],
        out_specs=pl.BlockSpec((BLK, NY), lambda i: (i, 0)),
        out_shape=jax.ShapeDtypeStruct((B, NY), jnp.float32),
        scratch_shapes=[pltpu.VMEM((BLK, NX + NH), jnp.bfloat16)],
    )(x, w_xh, w_cat)


def kernel(x, w_xy, w_xh, w_hy, k_y, k_h):
    w_cat = jnp.concatenate(
        [w_xy.astype(jnp.bfloat16), -w_hy.astype(jnp.bfloat16)], axis=0
    )
    return _wta(x, w_xh.astype(jnp.bfloat16), w_cat)


# speedup vs baseline: 20.3723x; 1.2057x over previous
import jax
import jax.numpy as jnp
from jax.experimental import pallas as pl
from jax.experimental.pallas import tpu as pltpu


def _kth_threshold(a, k, lo, hi, presteps):
    def cond(carry):
        lo, hi = carry
        return jnp.max(hi - lo) > 0.0

    def step(lo, hi):
        mid = jnp.floor((lo + hi + 1.0) * 0.5)
        cnt = jnp.sum((a >= mid).astype(jnp.float32), axis=-1, keepdims=True)
        ge = cnt >= k
        return jnp.where(ge, mid, lo), jnp.where(ge, hi, mid - 1.0)

    def body(carry):
        return step(*step(*carry))

    for _ in range(presteps):
        lo, hi = step(lo, hi)
    lo, hi = jax.lax.while_loop(cond, body, (lo, hi))
    return lo


def _wta_block(x_ref, wxh_ref, wcat_ref, y_ref, cat_ref):
    x = x_ref[...].astype(jnp.bfloat16)
    a_h = jnp.dot(x, wxh_ref[...], preferred_element_type=jnp.float32)
    thr_h = _kth_threshold(
        a_h, 13, jnp.zeros_like(a_h[:, :1]), jnp.max(a_h, axis=-1, keepdims=True),
        presteps=4,
    )
    h = jnp.where(a_h >= thr_h, a_h, 0.0).astype(jnp.bfloat16)
    cat_ref[:, : x.shape[1]] = x
    cat_ref[:, x.shape[1] :] = h
    a_y = jnp.dot(cat_ref[...], wcat_ref[...], preferred_element_type=jnp.float32)
    thr_y = _kth_threshold(
        a_y,
        51,
        jnp.min(a_y, axis=-1, keepdims=True),
        jnp.max(a_y, axis=-1, keepdims=True),
        presteps=6,
    )
    y_ref[...] = jnp.where(a_y >= thr_y, a_y, 0.0)


@jax.jit
def _wta(x, w_xh, w_cat):
    B, NX = x.shape
    NH = w_xh.shape[1]
    NY = w_cat.shape[1]
    BLK = 1024
    grid = (B // BLK,)
    return pl.pallas_call(
        _wta_block,
        grid=grid,
        in_specs=[
            pl.BlockSpec((BLK, NX), lambda i: (i, 0)),
            pl.BlockSpec((NX, NH), lambda i: (0, 0)),
            pl.BlockSpec((NX + NH, NY), lambda i: (0, 0)),
        ],
        out_specs=pl.BlockSpec((BLK, NY), lambda i: (i, 0)),
        out_shape=jax.ShapeDtypeStruct((B, NY), jnp.float32),
        scratch_shapes=[pltpu.VMEM((BLK, NX + NH), jnp.bfloat16)],
    )(x, w_xh, w_cat)


def kernel(x, w_xy, w_xh, w_hy, k_y, k_h):
    w_cat = jnp.concatenate(
        [w_xy.astype(jnp.bfloat16), -w_hy.astype(jnp.bfloat16)], axis=0
    )
    return _wta(x, w_xh.astype(jnp.bfloat16), w_cat)
